# baseline (speedup 1.0000x reference)
import functools
import math

import jax
import jax.numpy as jnp
from jax.experimental import pallas as pl
from jax.experimental.pallas import tpu as pltpu

HEADS = 12
TOPK = 8


def _ln_qkv_kernel(x_ref, s_ref, b_ref, w_ref, qkv_ref):
    x = x_ref[...]
    mu = jnp.mean(x, axis=-1, keepdims=True)
    var = jnp.mean((x - mu) ** 2, axis=-1, keepdims=True)
    h = (x - mu) / jnp.sqrt(var + 1e-5) * s_ref[...] + b_ref[...]
    qkv_ref[...] = jnp.dot(h, w_ref[...], preferred_element_type=jnp.float32)


def _attn_kernel(q_ref, k_ref, v_ref, o_ref, m_ref, l_ref, acc_ref, *,
                 sm_scale):
    iq, ik = pl.program_id(1), pl.program_id(2)

    def update(masked):
        q = q_ref[0] * sm_scale
        s = jax.lax.dot_general(q, k_ref[0], (((1,), (1,)), ((), ())),
                                preferred_element_type=jnp.float32)
        if masked:
            rows = jax.lax.broadcasted_iota(jnp.int32, s.shape, 0)
            cols = jax.lax.broadcasted_iota(jnp.int32, s.shape, 1)
            s = jnp.where(cols <= rows, s, -1e9)
        m = m_ref[:, :1]
        mj = jnp.maximum(m, jnp.max(s, axis=-1, keepdims=True))
        p = jnp.exp(s - mj)
        alpha = jnp.exp(m - mj)
        lj = l_ref[:, :1] * alpha + jnp.sum(p, axis=-1, keepdims=True)
        accj = acc_ref[...] * alpha + jnp.dot(
            p, v_ref[0], preferred_element_type=jnp.float32)
        m_ref[:, :1] = mj
        l_ref[:, :1] = lj
        acc_ref[...] = accj
        return lj, accj

    @pl.when(ik == 0)
    def _():
        m_ref[...] = jnp.full_like(m_ref, -1e30)
        l_ref[...] = jnp.zeros_like(l_ref)
        acc_ref[...] = jnp.zeros_like(acc_ref)

    @pl.when(ik < iq)
    def _():
        update(masked=False)

    @pl.when(ik == iq)
    def _():
        lj, accj = update(masked=True)
        o_ref[0] = accj / lj


def _post_kernel(attn_ref, wo_ref, x_ref, s_ref, b_ref, rw_ref, rb_ref,
                 x2_ref, h2_ref, g_ref, *, topk):
    x2 = x_ref[...] + jnp.dot(attn_ref[...], wo_ref[...],
                              preferred_element_type=jnp.float32)
    x2_ref[...] = x2
    mu = jnp.mean(x2, axis=-1, keepdims=True)
    var = jnp.mean((x2 - mu) ** 2, axis=-1, keepdims=True)
    h2 = (x2 - mu) / jnp.sqrt(var + 1e-5) * s_ref[...] + b_ref[...]
    h2_ref[...] = h2
    logits = jnp.dot(h2, rw_ref[...], preferred_element_type=jnp.float32)
    logits = logits + rb_ref[...]
    lanes = jax.lax.broadcasted_iota(jnp.int32, logits.shape, 1)
    work = logits
    g = jnp.zeros_like(logits)
    sumexp = jnp.zeros_like(logits[:, :1])
    v0 = jnp.max(work, axis=-1, keepdims=True)
    for _ in range(topk):
        vk = jnp.max(work, axis=-1, keepdims=True)
        hit = work == vk
        idx = jnp.min(jnp.where(hit, lanes, jnp.int32(2**30)),
                      axis=-1, keepdims=True)
        onehot = lanes == idx
        ek = jnp.exp(vk - v0)
        g = g + jnp.where(onehot, ek, 0.0)
        sumexp = sumexp + ek
        work = jnp.where(onehot, jnp.float32(-1e30), work)
    g_ref[...] = g / sumexp


def _moe_kernel(h_ref, g_ref, x2_ref, bank_ref, o_ref):
    e = pl.program_id(0)

    @pl.when(e == 0)
    def _():
        o_ref[...] = x2_ref[...]

    lanes = jax.lax.broadcasted_iota(jnp.int32, g_ref.shape, 1)
    gcol = jnp.sum(jnp.where(lanes == e, g_ref[...], 0.0),
                   axis=1, keepdims=True)
    y = jax.lax.dot_general(h_ref[...].astype(jnp.bfloat16),
                            bank_ref[0].astype(jnp.bfloat16),
                            (((1,), (1,)), ((), ())),
                            preferred_element_type=jnp.float32)
    o_ref[...] = o_ref[...] + y * gcol


def kernel(x, ln1_scale, ln1_bias, ln2_scale, ln2_bias, wq, wk, wv, wo,
           router_w, router_b, bank):
    b, s, d = x.shape
    e_num = router_w.shape[1]
    heads = HEADS
    dh = d // heads
    x2d = x.reshape(s, d)
    bt = min(256, s)
    nb = s // bt

    wcat = jnp.concatenate([wq, wk, wv], axis=1)
    qkv = pl.pallas_call(
        _ln_qkv_kernel,
        grid=(nb,),
        in_specs=[
            pl.BlockSpec((bt, d), lambda i: (i, 0)),
            pl.BlockSpec((1, d), lambda i: (0, 0)),
            pl.BlockSpec((1, d), lambda i: (0, 0)),
            pl.BlockSpec((d, 3 * d), lambda i: (0, 0)),
        ],
        out_specs=pl.BlockSpec((bt, 3 * d), lambda i: (i, 0)),
        out_shape=jax.ShapeDtypeStruct((s, 3 * d), jnp.float32),
    )(x2d, ln1_scale.reshape(1, d), ln1_bias.reshape(1, d), wcat)

    qkvh = qkv.reshape(s, 3, heads, dh).transpose(1, 2, 0, 3)
    q, k, v = qkvh[0], qkvh[1], qkvh[2]

    bq = min(256, s)
    nq = s // bq
    attn = pl.pallas_call(
        functools.partial(_attn_kernel, sm_scale=1.0 / math.sqrt(dh)),
        grid=(heads, nq, nq),
        in_specs=[
            pl.BlockSpec((1, bq, dh), lambda h, i, j: (h, i, 0)),
            pl.BlockSpec((1, bq, dh), lambda h, i, j: (h, j, 0)),
            pl.BlockSpec((1, bq, dh), lambda h, i, j: (h, j, 0)),
        ],
        out_specs=pl.BlockSpec((1, bq, dh), lambda h, i, j: (h, i, 0)),
        out_shape=jax.ShapeDtypeStruct((heads, s, dh), jnp.float32),
        scratch_shapes=[
            pltpu.VMEM((bq, 128), jnp.float32),
            pltpu.VMEM((bq, 128), jnp.float32),
            pltpu.VMEM((bq, dh), jnp.float32),
        ],
        compiler_params=pltpu.CompilerParams(
            dimension_semantics=("parallel", "parallel", "arbitrary")),
    )(q, k, v)
    attn2d = attn.transpose(1, 0, 2).reshape(s, d)

    x2, h2, g = pl.pallas_call(
        functools.partial(_post_kernel, topk=TOPK),
        grid=(nb,),
        in_specs=[
            pl.BlockSpec((bt, d), lambda i: (i, 0)),
            pl.BlockSpec((d, d), lambda i: (0, 0)),
            pl.BlockSpec((bt, d), lambda i: (i, 0)),
            pl.BlockSpec((1, d), lambda i: (0, 0)),
            pl.BlockSpec((1, d), lambda i: (0, 0)),
            pl.BlockSpec((d, e_num), lambda i: (0, 0)),
            pl.BlockSpec((1, e_num), lambda i: (0, 0)),
        ],
        out_specs=[
            pl.BlockSpec((bt, d), lambda i: (i, 0)),
            pl.BlockSpec((bt, d), lambda i: (i, 0)),
            pl.BlockSpec((bt, e_num), lambda i: (i, 0)),
        ],
        out_shape=[
            jax.ShapeDtypeStruct((s, d), jnp.float32),
            jax.ShapeDtypeStruct((s, d), jnp.float32),
            jax.ShapeDtypeStruct((s, e_num), jnp.float32),
        ],
    )(attn2d, wo, x2d, ln2_scale.reshape(1, d), ln2_bias.reshape(1, d),
      router_w, router_b.reshape(1, e_num))

    out = pl.pallas_call(
        _moe_kernel,
        grid=(e_num,),
        in_specs=[
            pl.BlockSpec((s, d), lambda e: (0, 0)),
            pl.BlockSpec((s, e_num), lambda e: (0, 0)),
            pl.BlockSpec((s, d), lambda e: (0, 0)),
            pl.BlockSpec((1, d, d), lambda e: (e, 0, 0)),
        ],
        out_specs=pl.BlockSpec((s, d), lambda e: (0, 0)),
        out_shape=jax.ShapeDtypeStruct((s, d), jnp.float32),
        compiler_params=pltpu.CompilerParams(
            dimension_semantics=("arbitrary",)),
    )(h2, g, x2, bank)

    return out.reshape(b, s, d)


# speedup vs baseline: 1.7204x; 1.7204x over previous
import functools
import math

import jax
import jax.numpy as jnp
from jax.experimental import pallas as pl
from jax.experimental.pallas import tpu as pltpu

HEADS = 12
TOPK = 8


def _ln_qkv_kernel(x_ref, s_ref, b_ref, w_ref, qkv_ref):
    x = x_ref[...]
    mu = jnp.mean(x, axis=-1, keepdims=True)
    var = jnp.mean((x - mu) ** 2, axis=-1, keepdims=True)
    h = (x - mu) / jnp.sqrt(var + 1e-5) * s_ref[...] + b_ref[...]
    qkv_ref[...] = jnp.dot(h, w_ref[...], preferred_element_type=jnp.float32)


def _attn_kernel(q_ref, k_ref, v_ref, o_ref, *, sm_scale, bq):
    iq = pl.program_id(1)
    q = q_ref[0] * sm_scale
    s = jax.lax.dot_general(q, k_ref[0], (((1,), (1,)), ((), ())),
                            preferred_element_type=jnp.float32)
    rows = iq * bq + jax.lax.broadcasted_iota(jnp.int32, s.shape, 0)
    cols = jax.lax.broadcasted_iota(jnp.int32, s.shape, 1)
    s = jnp.where(cols <= rows, s, -1e9)
    m = jnp.max(s, axis=-1, keepdims=True)
    p = jnp.exp(s - m)
    l = jnp.sum(p, axis=-1, keepdims=True)
    o_ref[0] = jnp.dot(p, v_ref[0], preferred_element_type=jnp.float32) / l


def _post_kernel(attn_ref, wo_ref, x_ref, s_ref, b_ref, rw_ref, rb_ref,
                 x2_ref, h2_ref, g_ref, *, topk):
    x2 = x_ref[...] + jnp.dot(attn_ref[...], wo_ref[...],
                              preferred_element_type=jnp.float32)
    x2_ref[...] = x2
    mu = jnp.mean(x2, axis=-1, keepdims=True)
    var = jnp.mean((x2 - mu) ** 2, axis=-1, keepdims=True)
    h2 = (x2 - mu) / jnp.sqrt(var + 1e-5) * s_ref[...] + b_ref[...]
    h2_ref[...] = h2
    logits = jnp.dot(h2, rw_ref[...], preferred_element_type=jnp.float32)
    logits = logits + rb_ref[...]
    lanes = jax.lax.broadcasted_iota(jnp.int32, logits.shape, 1)
    work = logits
    g = jnp.zeros_like(logits)
    sumexp = jnp.zeros_like(logits[:, :1])
    v0 = jnp.max(work, axis=-1, keepdims=True)
    for _ in range(topk):
        vk = jnp.max(work, axis=-1, keepdims=True)
        hit = work == vk
        idx = jnp.min(jnp.where(hit, lanes, jnp.int32(2**30)),
                      axis=-1, keepdims=True)
        onehot = lanes == idx
        ek = jnp.exp(vk - v0)
        g = g + jnp.where(onehot, ek, 0.0)
        sumexp = sumexp + ek
        work = jnp.where(onehot, jnp.float32(-1e30), work)
    g_ref[...] = g / sumexp


def _moe_kernel(h_ref, g_ref, x2_ref, bank_ref, o_ref):
    e = pl.program_id(0)

    @pl.when(e == 0)
    def _():
        o_ref[...] = x2_ref[...]

    lanes = jax.lax.broadcasted_iota(jnp.int32, g_ref.shape, 1)
    gcol = jnp.sum(jnp.where(lanes == e, g_ref[...], 0.0),
                   axis=1, keepdims=True)
    y = jax.lax.dot_general(h_ref[...].astype(jnp.bfloat16),
                            bank_ref[0].astype(jnp.bfloat16),
                            (((1,), (1,)), ((), ())),
                            preferred_element_type=jnp.float32)
    o_ref[...] = o_ref[...] + y * gcol


def kernel(x, ln1_scale, ln1_bias, ln2_scale, ln2_bias, wq, wk, wv, wo,
           router_w, router_b, bank):
    b, s, d = x.shape
    e_num = router_w.shape[1]
    heads = HEADS
    dh = d // heads
    x2d = x.reshape(s, d)
    bt = min(256, s)
    nb = s // bt

    wcat = jnp.concatenate([wq, wk, wv], axis=1)
    qkv = pl.pallas_call(
        _ln_qkv_kernel,
        grid=(nb,),
        in_specs=[
            pl.BlockSpec((bt, d), lambda i: (i, 0)),
            pl.BlockSpec((1, d), lambda i: (0, 0)),
            pl.BlockSpec((1, d), lambda i: (0, 0)),
            pl.BlockSpec((d, 3 * d), lambda i: (0, 0)),
        ],
        out_specs=pl.BlockSpec((bt, 3 * d), lambda i: (i, 0)),
        out_shape=jax.ShapeDtypeStruct((s, 3 * d), jnp.float32),
    )(x2d, ln1_scale.reshape(1, d), ln1_bias.reshape(1, d), wcat)

    qkvh = qkv.reshape(s, 3, heads, dh).transpose(1, 2, 0, 3)
    q, k, v = qkvh[0], qkvh[1], qkvh[2]

    bq = min(256, s)
    attn = pl.pallas_call(
        functools.partial(_attn_kernel, sm_scale=1.0 / math.sqrt(dh), bq=bq),
        grid=(heads, s // bq),
        in_specs=[
            pl.BlockSpec((1, bq, dh), lambda h, i: (h, i, 0)),
            pl.BlockSpec((1, s, dh), lambda h, i: (h, 0, 0)),
            pl.BlockSpec((1, s, dh), lambda h, i: (h, 0, 0)),
        ],
        out_specs=pl.BlockSpec((1, bq, dh), lambda h, i: (h, i, 0)),
        out_shape=jax.ShapeDtypeStruct((heads, s, dh), jnp.float32),
    )(q, k, v)
    attn2d = attn.transpose(1, 0, 2).reshape(s, d)

    x2, h2, g = pl.pallas_call(
        functools.partial(_post_kernel, topk=TOPK),
        grid=(nb,),
        in_specs=[
            pl.BlockSpec((bt, d), lambda i: (i, 0)),
            pl.BlockSpec((d, d), lambda i: (0, 0)),
            pl.BlockSpec((bt, d), lambda i: (i, 0)),
            pl.BlockSpec((1, d), lambda i: (0, 0)),
            pl.BlockSpec((1, d), lambda i: (0, 0)),
            pl.BlockSpec((d, e_num), lambda i: (0, 0)),
            pl.BlockSpec((1, e_num), lambda i: (0, 0)),
        ],
        out_specs=[
            pl.BlockSpec((bt, d), lambda i: (i, 0)),
            pl.BlockSpec((bt, d), lambda i: (i, 0)),
            pl.BlockSpec((bt, e_num), lambda i: (i, 0)),
        ],
        out_shape=[
            jax.ShapeDtypeStruct((s, d), jnp.float32),
            jax.ShapeDtypeStruct((s, d), jnp.float32),
            jax.ShapeDtypeStruct((s, e_num), jnp.float32),
        ],
    )(attn2d, wo, x2d, ln2_scale.reshape(1, d), ln2_bias.reshape(1, d),
      router_w, router_b.reshape(1, e_num))

    out = pl.pallas_call(
        _moe_kernel,
        grid=(e_num,),
        in_specs=[
            pl.BlockSpec((s, d), lambda e: (0, 0)),
            pl.BlockSpec((s, e_num), lambda e: (0, 0)),
            pl.BlockSpec((s, d), lambda e: (0, 0)),
            pl.BlockSpec((1, d, d), lambda e: (e, 0, 0)),
        ],
        out_specs=pl.BlockSpec((s, d), lambda e: (0, 0)),
        out_shape=jax.ShapeDtypeStruct((s, d), jnp.float32),
        compiler_params=pltpu.CompilerParams(
            dimension_semantics=("arbitrary",)),
    )(h2, g, x2, bank)

    return out.reshape(b, s, d)
